# baseline (speedup 1.0000x reference)
import jax
import jax.numpy as jnp
from jax import lax
from jax.experimental import pallas as pl
from jax.experimental.pallas import tpu as pltpu

F32 = jnp.float32
BF16 = jnp.bfloat16


def _pow_m34(u):
    r = lax.rsqrt(u)
    return r * jnp.sqrt(r)


def _lrn(y, band):
    sq = (y * y).astype(BF16)
    div = jnp.dot(sq, band, preferred_element_type=F32)
    u = 2.0 + 1e-4 * div
    return y * _pow_m34(u)


def _maxpool3s2(y):
    H, W, C = y.shape
    OH, OW = (H - 3) // 2 + 1, (W - 3) // 2 + 1
    yp = jnp.pad(y, ((0, H % 2), (0, W % 2), (0, 0)))
    a = yp.reshape((H + H % 2) // 2, 2, W + W % 2, C)
    r = jnp.maximum(jnp.maximum(a[0:OH, 0], a[0:OH, 1]), a[1:OH + 1, 0])
    b = r.reshape(OH, (W + W % 2) // 2, 2, C)
    return jnp.maximum(jnp.maximum(b[:, 0:OW, 0], b[:, 0:OW, 1]),
                       b[:, 1:OW + 1, 0])


def _stage_a_kernel(x2_ref, w_ref, b_ref, band_ref, o_ref):
    for n in range(x2_ref.shape[0]):
        pieces = []
        for kh in range(11):
            for g in range(3):
                base = (kh // 4) * 57 + g
                pieces.append(x2_ref[n, kh % 4, :, base:base + 3135])
        pt = jnp.concatenate(pieces, axis=0)
        y = lax.dot_general(pt, w_ref[...], (((0,), (0,)), ((), ())),
                            preferred_element_type=F32)
        y = jnp.maximum(y + b_ref[...], 0.0)
        y = _lrn(y, band_ref[...]).astype(BF16)
        y = y.reshape(55, 57, 128)
        o_ref[n] = _maxpool3s2(y[:, :55, :])


def _stage_b_kernel(h_ref, w_ref, b_ref, band_ref, o_ref):
    for n in range(h_ref.shape[0]):
        xp = jnp.pad(h_ref[n], ((2, 2), (2, 2), (0, 0)))
        y = None
        for kh in range(5):
            for kw in range(5):
                t = kh * 5 + kw
                p = xp[kh:kh + 27, kw:kw + 27, :].reshape(729, 128)
                d = jnp.dot(p, w_ref[t], preferred_element_type=F32)
                y = d if y is None else y + d
        y = jnp.maximum(y + b_ref[...], 0.0)
        y = _lrn(y, band_ref[...]).astype(BF16)
        o_ref[n] = _maxpool3s2(y.reshape(27, 27, 256))


def _conv3x3(xp, w_ref, b_ref):
    H = xp.shape[0] - 2
    y = None
    for kh in range(3):
        for kw in range(3):
            p = xp[kh:kh + H, kw:kw + H, :].reshape(H * H, xp.shape[2])
            d = jnp.dot(p, w_ref[kh * 3 + kw], preferred_element_type=F32)
            y = d if y is None else y + d
    return jnp.maximum(y + b_ref[...], 0.0)


def _stage_c_kernel(h_ref, w3_ref, b3_ref, w4_ref, b4_ref, w5_ref, b5_ref,
                    o_ref):
    for n in range(h_ref.shape[0]):
        xp = jnp.pad(h_ref[n], ((1, 1), (1, 1), (0, 0)))
        h3 = _conv3x3(xp, w3_ref, b3_ref).astype(BF16)
        xp4 = jnp.pad(h3.reshape(13, 13, 384), ((1, 1), (1, 1), (0, 0)))
        h4 = _conv3x3(xp4, w4_ref, b4_ref).astype(BF16)
        xp5 = jnp.pad(h4.reshape(13, 13, 384), ((1, 1), (1, 1), (0, 0)))
        h5 = _conv3x3(xp5, w5_ref, b5_ref).astype(BF16)
        pooled = _maxpool3s2(h5.reshape(13, 13, 256))
        o_ref[n] = pooled.reshape(36, 256).T


def _fc_kernel(h_ref, w_ref, b_ref, o_ref):
    y = lax.dot_general(h_ref[...], w_ref[...], (((1,), (1,)), ((), ())),
                        preferred_element_type=F32)
    o_ref[...] = jnp.maximum(y + b_ref[...], 0.0)


def _band(c):
    i = lax.broadcasted_iota(jnp.int32, (c, c), 0)
    j = lax.broadcasted_iota(jnp.int32, (c, c), 1)
    return (jnp.abs(i - j) <= 2).astype(BF16)


def kernel(x, W1, b1, W2, b2, W3, b3, W4, b4, W5, b5, Wf1, bf1, Wf2, bf2,
           Wf3, bf3):
    B = x.shape[0]
    cp = lambda sem, vmem: pltpu.CompilerParams(
        dimension_semantics=sem, vmem_limit_bytes=vmem)

    GA, GB, GC = 1, 1, 4
    xb = x.astype(BF16)
    xs = jnp.pad(xb, ((0, 0), (0, 0), (2, 2), (2, 2)))
    x6 = xs.reshape(B, 3, 57, 4, 57, 4)
    X2 = x6.transpose(0, 3, 5, 1, 2, 4).reshape(B, 4, 12, 3249)
    X2 = jnp.pad(X2, ((0, 0), (0, 0), (0, 4), (0, 2)))

    W1p = jnp.pad(W1, ((0, 0), (0, 0), (0, 0), (0, 1)))
    W1r = W1p.reshape(96, 3, 11, 3, 4).transpose(2, 3, 4, 1, 0)
    W1k = jnp.pad(W1r.reshape(11, 3, 12, 96),
                  ((0, 0), (0, 0), (0, 4), (0, 32))).reshape(528, 128)
    W1k = W1k.astype(BF16)

    band128 = _band(128)
    h1 = pl.pallas_call(
        _stage_a_kernel,
        grid=(B // GA,),
        in_specs=[
            pl.BlockSpec((GA, 4, 16, 3251), lambda i: (i, 0, 0, 0)),
            pl.BlockSpec((528, 128), lambda i: (0, 0)),
            pl.BlockSpec((1, 128), lambda i: (0, 0)),
            pl.BlockSpec((128, 128), lambda i: (0, 0)),
        ],
        out_specs=pl.BlockSpec((GA, 27, 27, 128), lambda i: (i, 0, 0, 0)),
        out_shape=jax.ShapeDtypeStruct((B, 27, 27, 128), BF16),
        compiler_params=cp(("arbitrary",), 56 * 1024 * 1024),
    )(X2, W1k, jnp.pad(b1, (0, 32)).reshape(1, 128), band128)

    W2k = W2.transpose(2, 3, 1, 0).reshape(25, 96, 256)
    W2k = jnp.pad(W2k, ((0, 0), (0, 32), (0, 0))).astype(BF16)
    band256 = _band(256)
    h2 = pl.pallas_call(
        _stage_b_kernel,
        grid=(B // GB,),
        in_specs=[
            pl.BlockSpec((GB, 27, 27, 128), lambda i: (i, 0, 0, 0)),
            pl.BlockSpec((25, 128, 256), lambda i: (0, 0, 0)),
            pl.BlockSpec((1, 256), lambda i: (0, 0)),
            pl.BlockSpec((256, 256), lambda i: (0, 0)),
        ],
        out_specs=pl.BlockSpec((GB, 13, 13, 256), lambda i: (i, 0, 0, 0)),
        out_shape=jax.ShapeDtypeStruct((B, 13, 13, 256), BF16),
        compiler_params=cp(("arbitrary",), 56 * 1024 * 1024),
    )(h1, W2k, b2.reshape(1, 256), band256)

    W3k = W3.transpose(2, 3, 1, 0).reshape(9, 256, 384).astype(BF16)
    W4k = W4.transpose(2, 3, 1, 0).reshape(9, 384, 384).astype(BF16)
    W5k = W5.transpose(2, 3, 1, 0).reshape(9, 384, 256).astype(BF16)
    h5 = pl.pallas_call(
        _stage_c_kernel,
        grid=(B // GC,),
        in_specs=[
            pl.BlockSpec((GC, 13, 13, 256), lambda i: (i, 0, 0, 0)),
            pl.BlockSpec((9, 256, 384), lambda i: (0, 0, 0)),
            pl.BlockSpec((1, 384), lambda i: (0, 0)),
            pl.BlockSpec((9, 384, 384), lambda i: (0, 0, 0)),
            pl.BlockSpec((1, 384), lambda i: (0, 0)),
            pl.BlockSpec((9, 384, 256), lambda i: (0, 0, 0)),
            pl.BlockSpec((1, 256), lambda i: (0, 0)),
        ],
        out_specs=pl.BlockSpec((GC, 256, 36), lambda i: (i, 0, 0)),
        out_shape=jax.ShapeDtypeStruct((B, 256, 36), BF16),
        compiler_params=cp(("arbitrary",), 56 * 1024 * 1024),
    )(h2, W3k, b3.reshape(1, 384), W4k, b4.reshape(1, 384), W5k,
      b5.reshape(1, 256))

    hf = h5.reshape(B, 9216).astype(F32)

    def fc(h, W, b, nblk, vmem):
        N, K = W.shape
        return pl.pallas_call(
            _fc_kernel,
            grid=(N // nblk,),
            in_specs=[
                pl.BlockSpec((h.shape[0], K), lambda j: (0, 0)),
                pl.BlockSpec((nblk, K), lambda j: (j, 0)),
                pl.BlockSpec((1, nblk), lambda j: (0, j)),
            ],
            out_specs=pl.BlockSpec((h.shape[0], nblk), lambda j: (0, j)),
            out_shape=jax.ShapeDtypeStruct((h.shape[0], N), F32),
            compiler_params=cp(("arbitrary",), vmem),
        )(h, W, b.reshape(1, N))

    g1 = fc(hf, Wf1, bf1, 512, 56 * 1024 * 1024)
    g2 = fc(g1, Wf2, bf2, 512, 48 * 1024 * 1024)
    out = fc(g2, Wf3, bf3, 1000, 48 * 1024 * 1024)
    return out


# speedup vs baseline: 6.2682x; 6.2682x over previous
import jax
import jax.numpy as jnp
from jax import lax
from jax.experimental import pallas as pl
from jax.experimental.pallas import tpu as pltpu

F32 = jnp.float32
BF16 = jnp.bfloat16


def _pow_m34(u):
    r = lax.rsqrt(u)
    return r * jnp.sqrt(r)


def _lrn(y, band):
    sq = (y * y).astype(BF16)
    div = jnp.dot(sq, band, preferred_element_type=F32)
    u = 2.0 + 1e-4 * div
    return y * _pow_m34(u)


def _maxpool3s2(y):
    H, W, C = y.shape
    OH, OW = (H - 3) // 2 + 1, (W - 3) // 2 + 1
    yp = jnp.pad(y, ((0, H % 2), (0, W % 2), (0, 0)))
    a = yp.reshape((H + H % 2) // 2, 2, W + W % 2, C)
    r = jnp.maximum(jnp.maximum(a[0:OH, 0], a[0:OH, 1]), a[1:OH + 1, 0])
    b = r.reshape(OH, (W + W % 2) // 2, 2, C)
    return jnp.maximum(jnp.maximum(b[:, 0:OW, 0], b[:, 0:OW, 1]),
                       b[:, 1:OW + 1, 0])


def _stage_a_kernel(x2_ref, w_ref, b_ref, band_ref, o_ref):
    for n in range(x2_ref.shape[0]):
        pieces = []
        for kh in range(11):
            for g in range(3):
                base = (kh // 4) * 57 + g
                pieces.append(x2_ref[n, kh % 4, :, base:base + 3135])
        pt = jnp.concatenate(pieces, axis=0).astype(BF16)
        y = lax.dot_general(pt, w_ref[...], (((0,), (0,)), ((), ())),
                            preferred_element_type=F32)
        y = jnp.maximum(y + b_ref[...], 0.0)
        y = _lrn(y, band_ref[...]).astype(BF16)
        y = y.reshape(55, 57, 128)
        o_ref[n] = _maxpool3s2(y[:, :55, :])


def _stage_b_kernel(h_ref, w_ref, b_ref, band_ref, o_ref):
    for n in range(h_ref.shape[0]):
        xp = jnp.pad(h_ref[n], ((2, 2), (2, 2), (0, 0)))
        y = None
        for kh in range(5):
            for kw in range(5):
                t = kh * 5 + kw
                p = xp[kh:kh + 27, kw:kw + 27, :].reshape(729, 128)
                d = jnp.dot(p, w_ref[t], preferred_element_type=F32)
                y = d if y is None else y + d
        y = jnp.maximum(y + b_ref[...], 0.0)
        y = _lrn(y, band_ref[...]).astype(BF16)
        o_ref[n] = _maxpool3s2(y.reshape(27, 27, 256))


def _conv3x3(xp, w_ref, b_ref):
    H = xp.shape[0] - 2
    y = None
    for kh in range(3):
        for kw in range(3):
            p = xp[kh:kh + H, kw:kw + H, :].reshape(H * H, xp.shape[2])
            d = jnp.dot(p, w_ref[kh * 3 + kw], preferred_element_type=F32)
            y = d if y is None else y + d
    return jnp.maximum(y + b_ref[...], 0.0)


def _stage_c_kernel(h_ref, w3_ref, b3_ref, w4_ref, b4_ref, w5_ref, b5_ref,
                    o_ref):
    for n in range(h_ref.shape[0]):
        xp = jnp.pad(h_ref[n], ((1, 1), (1, 1), (0, 0)))
        h3 = _conv3x3(xp, w3_ref, b3_ref).astype(BF16)
        xp4 = jnp.pad(h3.reshape(13, 13, 384), ((1, 1), (1, 1), (0, 0)))
        h4 = _conv3x3(xp4, w4_ref, b4_ref).astype(BF16)
        xp5 = jnp.pad(h4.reshape(13, 13, 384), ((1, 1), (1, 1), (0, 0)))
        h5 = _conv3x3(xp5, w5_ref, b5_ref).astype(BF16)
        pooled = _maxpool3s2(h5.reshape(13, 13, 256))
        o_ref[n] = pooled.reshape(36, 256).T


def _fc_kernel(h_ref, w_ref, b_ref, o_ref):
    y = lax.dot_general(h_ref[...], w_ref[...], (((1,), (1,)), ((), ())),
                        preferred_element_type=F32)
    o_ref[...] = jnp.maximum(y + b_ref[...], 0.0)


def _band(c):
    i = lax.broadcasted_iota(jnp.int32, (c, c), 0)
    j = lax.broadcasted_iota(jnp.int32, (c, c), 1)
    return (jnp.abs(i - j) <= 2).astype(BF16)


def kernel(x, W1, b1, W2, b2, W3, b3, W4, b4, W5, b5, Wf1, bf1, Wf2, bf2,
           Wf3, bf3):
    B = x.shape[0]
    cp = lambda sem, vmem: pltpu.CompilerParams(
        dimension_semantics=sem, vmem_limit_bytes=vmem)

    GA, GB, GC = 1, 1, 4
    xs = jnp.pad(x, ((0, 0), (0, 0), (2, 2), (2, 2)))
    x6 = xs.reshape(B, 3, 57, 4, 57, 4)
    X2 = x6.transpose(0, 3, 5, 1, 2, 4).reshape(B, 4, 12, 3249)
    X2 = jnp.pad(X2, ((0, 0), (0, 0), (0, 4), (0, 2)))

    W1p = jnp.pad(W1, ((0, 0), (0, 0), (0, 0), (0, 1)))
    W1r = W1p.reshape(96, 3, 11, 3, 4).transpose(2, 3, 4, 1, 0)
    W1k = jnp.pad(W1r.reshape(11, 3, 12, 96),
                  ((0, 0), (0, 0), (0, 4), (0, 32))).reshape(528, 128)
    W1k = W1k.astype(BF16)

    band128 = _band(128)
    h1 = pl.pallas_call(
        _stage_a_kernel,
        grid=(B // GA,),
        in_specs=[
            pl.BlockSpec((GA, 4, 16, 3251), lambda i: (i, 0, 0, 0)),
            pl.BlockSpec((528, 128), lambda i: (0, 0)),
            pl.BlockSpec((1, 128), lambda i: (0, 0)),
            pl.BlockSpec((128, 128), lambda i: (0, 0)),
        ],
        out_specs=pl.BlockSpec((GA, 27, 27, 128), lambda i: (i, 0, 0, 0)),
        out_shape=jax.ShapeDtypeStruct((B, 27, 27, 128), BF16),
        compiler_params=cp(("arbitrary",), 56 * 1024 * 1024),
    )(X2, W1k, jnp.pad(b1, (0, 32)).reshape(1, 128), band128)

    W2k = W2.transpose(2, 3, 1, 0).reshape(25, 96, 256)
    W2k = jnp.pad(W2k, ((0, 0), (0, 32), (0, 0))).astype(BF16)
    band256 = _band(256)
    h2 = pl.pallas_call(
        _stage_b_kernel,
        grid=(B // GB,),
        in_specs=[
            pl.BlockSpec((GB, 27, 27, 128), lambda i: (i, 0, 0, 0)),
            pl.BlockSpec((25, 128, 256), lambda i: (0, 0, 0)),
            pl.BlockSpec((1, 256), lambda i: (0, 0)),
            pl.BlockSpec((256, 256), lambda i: (0, 0)),
        ],
        out_specs=pl.BlockSpec((GB, 13, 13, 256), lambda i: (i, 0, 0, 0)),
        out_shape=jax.ShapeDtypeStruct((B, 13, 13, 256), BF16),
        compiler_params=cp(("arbitrary",), 56 * 1024 * 1024),
    )(h1, W2k, b2.reshape(1, 256), band256)

    W3k = W3.transpose(2, 3, 1, 0).reshape(9, 256, 384).astype(BF16)
    W4k = W4.transpose(2, 3, 1, 0).reshape(9, 384, 384).astype(BF16)
    W5k = W5.transpose(2, 3, 1, 0).reshape(9, 384, 256).astype(BF16)
    h5 = pl.pallas_call(
        _stage_c_kernel,
        grid=(B // GC,),
        in_specs=[
            pl.BlockSpec((GC, 13, 13, 256), lambda i: (i, 0, 0, 0)),
            pl.BlockSpec((9, 256, 384), lambda i: (0, 0, 0)),
            pl.BlockSpec((1, 384), lambda i: (0, 0)),
            pl.BlockSpec((9, 384, 384), lambda i: (0, 0, 0)),
            pl.BlockSpec((1, 384), lambda i: (0, 0)),
            pl.BlockSpec((9, 384, 256), lambda i: (0, 0, 0)),
            pl.BlockSpec((1, 256), lambda i: (0, 0)),
        ],
        out_specs=pl.BlockSpec((GC, 256, 36), lambda i: (i, 0, 0)),
        out_shape=jax.ShapeDtypeStruct((B, 256, 36), BF16),
        compiler_params=cp(("arbitrary",), 56 * 1024 * 1024),
    )(h2, W3k, b3.reshape(1, 384), W4k, b4.reshape(1, 384), W5k,
      b5.reshape(1, 256))

    hf = h5.reshape(B, 9216).astype(F32)

    def fc(h, W, b, nblk, vmem):
        N, K = W.shape
        return pl.pallas_call(
            _fc_kernel,
            grid=(N // nblk,),
            in_specs=[
                pl.BlockSpec((h.shape[0], K), lambda j: (0, 0)),
                pl.BlockSpec((nblk, K), lambda j: (j, 0)),
                pl.BlockSpec((1, nblk), lambda j: (0, j)),
            ],
            out_specs=pl.BlockSpec((h.shape[0], nblk), lambda j: (0, j)),
            out_shape=jax.ShapeDtypeStruct((h.shape[0], N), F32),
            compiler_params=cp(("arbitrary",), vmem),
        )(h, W, b.reshape(1, N))

    g1 = fc(hf, Wf1, bf1, 512, 56 * 1024 * 1024)
    g2 = fc(g1, Wf2, bf2, 512, 48 * 1024 * 1024)
    out = fc(g2, Wf3, bf3, 1000, 48 * 1024 * 1024)
    return out


# speedup vs baseline: 6.7709x; 1.0802x over previous
import jax
import jax.numpy as jnp
from jax import lax
from jax.experimental import pallas as pl
from jax.experimental.pallas import tpu as pltpu

F32 = jnp.float32
BF16 = jnp.bfloat16


def _pow_m34(u):
    r = lax.rsqrt(u)
    return r * jnp.sqrt(r)


def _lrn(y, band):
    sq = (y * y).astype(BF16)
    div = jnp.dot(sq, band, preferred_element_type=F32)
    u = 2.0 + 1e-4 * div
    return y * _pow_m34(u)


def _maxpool3s2(y, OH=None, OW=None):
    H, W, C = y.shape
    if OH is None:
        OH, OW = (H - 3) // 2 + 1, (W - 3) // 2 + 1
    yp = jnp.pad(y, ((0, H % 2), (0, W % 2), (0, 0)))
    a = yp.reshape((H + H % 2) // 2, 2, W + W % 2, C)
    r = jnp.maximum(jnp.maximum(a[0:OH, 0], a[0:OH, 1]), a[1:OH + 1, 0])
    b = r.reshape(OH, (W + W % 2) // 2, 2, C)
    return jnp.maximum(jnp.maximum(b[:, 0:OW, 0], b[:, 0:OW, 1]),
                       b[:, 1:OW + 1, 0])


def _stage_a_kernel(x2_ref, w_ref, b_ref, band_ref, o_ref):
    for n in range(x2_ref.shape[0]):
        pieces = []
        for kh in range(11):
            for g in range(3):
                base = (kh // 4) * 57 + g
                pieces.append(x2_ref[n, kh % 4, :, base:base + 3135])
        pt = jnp.concatenate(pieces, axis=0).astype(BF16)
        y = lax.dot_general(pt, w_ref[...], (((0,), (0,)), ((), ())),
                            preferred_element_type=F32)
        y = jnp.maximum(y + b_ref[...], 0.0)
        y = _lrn(y, band_ref[...]).astype(BF16)
        y = y.reshape(55, 57, 128)
        o_ref[n] = _maxpool3s2(y[:, :55, :])


def _stage_b_kernel(h_ref, w_ref, b_ref, band_ref, o_ref):
    for n in range(h_ref.shape[0]):
        xp = jnp.pad(h_ref[n], ((2, 3), (2, 3), (0, 0)))
        xf = xp.reshape(1024, 128)
        F = jnp.concatenate([xf[kw:kw + 992, :] for kw in range(5)],
                            axis=1)
        y = None
        for kh in range(5):
            d = jnp.dot(F[kh * 32:kh * 32 + 864, :], w_ref[kh],
                        preferred_element_type=F32)
            y = d if y is None else y + d
        y = jnp.maximum(y + b_ref[...], 0.0)
        y = _lrn(y, band_ref[...]).astype(BF16)
        pooled = _maxpool3s2(y.reshape(27, 32, 256), 13, 13)
        o_ref[n] = jnp.pad(pooled, ((1, 2), (1, 2), (0, 0)))


def _conv3x3(xp, w_ref, b_ref):
    xf = xp.reshape(256, xp.shape[2])
    F = jnp.concatenate([xf[kw:kw + 240, :] for kw in range(3)],
                        axis=1)
    y = None
    for kh in range(3):
        d = jnp.dot(F[kh * 16:kh * 16 + 208, :], w_ref[kh],
                    preferred_element_type=F32)
        y = d if y is None else y + d
    return jnp.maximum(y + b_ref[...], 0.0)


def _repad(y, C):
    y3 = y.astype(BF16).reshape(13, 16, C)
    ow = lax.broadcasted_iota(jnp.int32, (1, 16, 1), 1)
    y3 = jnp.where(ow < 13, y3, jnp.bfloat16(0))
    return jnp.pad(y3, ((1, 2), (1, 2), (0, 0)))[:, 0:16, :]


def _stage_c_kernel(h_ref, w3_ref, b3_ref, w4_ref, b4_ref, w5_ref, b5_ref,
                    o_ref):
    for n in range(h_ref.shape[0]):
        h3 = _conv3x3(h_ref[n], w3_ref, b3_ref)
        h4 = _conv3x3(_repad(h3, 384), w4_ref, b4_ref)
        h5 = _conv3x3(_repad(h4, 384), w5_ref, b5_ref).astype(BF16)
        pooled = _maxpool3s2(h5.reshape(13, 16, 256), 6, 6)
        o_ref[n] = pooled.reshape(36, 256).T


def _fc_kernel(h_ref, w_ref, b_ref, o_ref):
    y = lax.dot_general(h_ref[...], w_ref[...], (((1,), (1,)), ((), ())),
                        preferred_element_type=F32)
    o_ref[...] = jnp.maximum(y + b_ref[...], 0.0)


def _band(c):
    i = lax.broadcasted_iota(jnp.int32, (c, c), 0)
    j = lax.broadcasted_iota(jnp.int32, (c, c), 1)
    return (jnp.abs(i - j) <= 2).astype(BF16)


def kernel(x, W1, b1, W2, b2, W3, b3, W4, b4, W5, b5, Wf1, bf1, Wf2, bf2,
           Wf3, bf3):
    B = x.shape[0]
    cp = lambda sem, vmem: pltpu.CompilerParams(
        dimension_semantics=sem, vmem_limit_bytes=vmem)

    GA, GB, GC = 1, 1, 4
    xs = jnp.pad(x, ((0, 0), (0, 0), (2, 2), (2, 2)))
    x6 = xs.reshape(B, 3, 57, 4, 57, 4)
    X2 = x6.transpose(0, 3, 5, 1, 2, 4).reshape(B, 4, 12, 3249)
    X2 = jnp.pad(X2, ((0, 0), (0, 0), (0, 4), (0, 2)))

    W1p = jnp.pad(W1, ((0, 0), (0, 0), (0, 0), (0, 1)))
    W1r = W1p.reshape(96, 3, 11, 3, 4).transpose(2, 3, 4, 1, 0)
    W1k = jnp.pad(W1r.reshape(11, 3, 12, 96),
                  ((0, 0), (0, 0), (0, 4), (0, 32))).reshape(528, 128)
    W1k = W1k.astype(BF16)

    band128 = _band(128)
    h1 = pl.pallas_call(
        _stage_a_kernel,
        grid=(B // GA,),
        in_specs=[
            pl.BlockSpec((GA, 4, 16, 3251), lambda i: (i, 0, 0, 0)),
            pl.BlockSpec((528, 128), lambda i: (0, 0)),
            pl.BlockSpec((1, 128), lambda i: (0, 0)),
            pl.BlockSpec((128, 128), lambda i: (0, 0)),
        ],
        out_specs=pl.BlockSpec((GA, 27, 27, 128), lambda i: (i, 0, 0, 0)),
        out_shape=jax.ShapeDtypeStruct((B, 27, 27, 128), BF16),
        compiler_params=cp(("arbitrary",), 56 * 1024 * 1024),
    )(X2, W1k, jnp.pad(b1, (0, 32)).reshape(1, 128), band128)

    W2k = jnp.pad(W2.transpose(2, 3, 1, 0), ((0, 0), (0, 0), (0, 32), (0, 0)))
    W2k = W2k.reshape(5, 640, 256).astype(BF16)
    band256 = _band(256)
    h2 = pl.pallas_call(
        _stage_b_kernel,
        grid=(B // GB,),
        in_specs=[
            pl.BlockSpec((GB, 27, 27, 128), lambda i: (i, 0, 0, 0)),
            pl.BlockSpec((5, 640, 256), lambda i: (0, 0, 0)),
            pl.BlockSpec((1, 256), lambda i: (0, 0)),
            pl.BlockSpec((256, 256), lambda i: (0, 0)),
        ],
        out_specs=pl.BlockSpec((GB, 16, 16, 256), lambda i: (i, 0, 0, 0)),
        out_shape=jax.ShapeDtypeStruct((B, 16, 16, 256), BF16),
        compiler_params=cp(("arbitrary",), 56 * 1024 * 1024),
    )(h1, W2k, b2.reshape(1, 256), band256)

    W3k = W3.transpose(2, 3, 1, 0).reshape(3, 768, 384).astype(BF16)
    W4k = W4.transpose(2, 3, 1, 0).reshape(3, 1152, 384).astype(BF16)
    W5k = W5.transpose(2, 3, 1, 0).reshape(3, 1152, 256).astype(BF16)
    h5 = pl.pallas_call(
        _stage_c_kernel,
        grid=(B // GC,),
        in_specs=[
            pl.BlockSpec((GC, 16, 16, 256), lambda i: (i, 0, 0, 0)),
            pl.BlockSpec((3, 768, 384), lambda i: (0, 0, 0)),
            pl.BlockSpec((1, 384), lambda i: (0, 0)),
            pl.BlockSpec((3, 1152, 384), lambda i: (0, 0, 0)),
            pl.BlockSpec((1, 384), lambda i: (0, 0)),
            pl.BlockSpec((3, 1152, 256), lambda i: (0, 0, 0)),
            pl.BlockSpec((1, 256), lambda i: (0, 0)),
        ],
        out_specs=pl.BlockSpec((GC, 256, 36), lambda i: (i, 0, 0)),
        out_shape=jax.ShapeDtypeStruct((B, 256, 36), BF16),
        compiler_params=cp(("arbitrary",), 56 * 1024 * 1024),
    )(h2, W3k, b3.reshape(1, 384), W4k, b4.reshape(1, 384), W5k,
      b5.reshape(1, 256))

    hf = h5.reshape(B, 9216).astype(F32)

    def fc(h, W, b, nblk, vmem):
        N, K = W.shape
        return pl.pallas_call(
            _fc_kernel,
            grid=(N // nblk,),
            in_specs=[
                pl.BlockSpec((h.shape[0], K), lambda j: (0, 0)),
                pl.BlockSpec((nblk, K), lambda j: (j, 0)),
                pl.BlockSpec((1, nblk), lambda j: (0, j)),
            ],
            out_specs=pl.BlockSpec((h.shape[0], nblk), lambda j: (0, j)),
            out_shape=jax.ShapeDtypeStruct((h.shape[0], N), F32),
            compiler_params=cp(("arbitrary",), vmem),
        )(h, W, b.reshape(1, N))

    g1 = fc(hf, Wf1, bf1, 512, 56 * 1024 * 1024)
    g2 = fc(g1, Wf2, bf2, 512, 48 * 1024 * 1024)
    out = fc(g2, Wf3, bf3, 1000, 48 * 1024 * 1024)
    return out


# speedup vs baseline: 7.6322x; 1.1272x over previous
import jax
import jax.numpy as jnp
from jax import lax
from jax.experimental import pallas as pl
from jax.experimental.pallas import tpu as pltpu

F32 = jnp.float32
BF16 = jnp.bfloat16


def _pow_m34(u):
    r = lax.rsqrt(u)
    return r * jnp.sqrt(r)


def _lrn(y, band):
    sq = (y * y).astype(BF16)
    div = jnp.dot(sq, band, preferred_element_type=F32)
    u = 2.0 + 1e-4 * div
    return y * _pow_m34(u)


def _maxpool3s2(y, OH=None, OW=None):
    H, W, C = y.shape
    if OH is None:
        OH, OW = (H - 3) // 2 + 1, (W - 3) // 2 + 1
    yp = jnp.pad(y, ((0, H % 2), (0, W % 2), (0, 0)))
    a = yp.reshape((H + H % 2) // 2, 2, W + W % 2, C)
    r = jnp.maximum(jnp.maximum(a[0:OH, 0], a[0:OH, 1]), a[1:OH + 1, 0])
    b = r.reshape(OH, (W + W % 2) // 2, 2, C)
    return jnp.maximum(jnp.maximum(b[:, 0:OW, 0], b[:, 0:OW, 1]),
                       b[:, 1:OW + 1, 0])


def _stage_a_kernel(x2_ref, w_ref, b_ref, band_ref, o_ref):
    for n in range(x2_ref.shape[0]):
        pieces = []
        for kh in range(11):
            for g in range(3):
                base = (kh // 4) * 64 + g
                pieces.append(x2_ref[n, kh % 4, :, base:base + 3584])
        pt = jnp.concatenate(pieces, axis=0).astype(BF16)
        y = lax.dot_general(pt, w_ref[...], (((0,), (0,)), ((), ())),
                            preferred_element_type=F32)
        y = jnp.maximum(y + b_ref[...], 0.0)
        y = _lrn(y, band_ref[...]).astype(BF16)
        y = y.reshape(56, 64, 128)
        o_ref[n] = _maxpool3s2(y, 27, 27)


def _stage_b_kernel(h_ref, w_ref, b_ref, band_ref, o_ref):
    for n in range(h_ref.shape[0]):
        xp = jnp.pad(h_ref[n], ((2, 3), (2, 3), (0, 0)))
        xf = xp.reshape(1024, 128)
        F = jnp.concatenate([xf[kw:kw + 992, :] for kw in range(5)],
                            axis=1)
        y = None
        for kh in range(5):
            d = jnp.dot(F[kh * 32:kh * 32 + 864, :], w_ref[kh],
                        preferred_element_type=F32)
            y = d if y is None else y + d
        y = jnp.maximum(y + b_ref[...], 0.0)
        y = _lrn(y, band_ref[...]).astype(BF16)
        pooled = _maxpool3s2(y.reshape(27, 32, 256), 13, 13)
        o_ref[n] = jnp.pad(pooled, ((1, 2), (1, 2), (0, 0)))


def _conv3x3(xp, w_ref, b_ref):
    xf = xp.reshape(256, xp.shape[2])
    F = jnp.concatenate([xf[kw:kw + 240, :] for kw in range(3)],
                        axis=1)
    y = None
    for kh in range(3):
        d = jnp.dot(F[kh * 16:kh * 16 + 208, :], w_ref[kh],
                    preferred_element_type=F32)
        y = d if y is None else y + d
    return jnp.maximum(y + b_ref[...], 0.0)


def _repad(y, C):
    y3 = y.astype(BF16).reshape(13, 16, C)
    ow = lax.broadcasted_iota(jnp.int32, (1, 16, 1), 1)
    y3 = jnp.where(ow < 13, y3, jnp.bfloat16(0))
    return jnp.pad(y3, ((1, 2), (1, 2), (0, 0)))[:, 0:16, :]


def _stage_c_kernel(h_ref, w3_ref, b3_ref, w4_ref, b4_ref, w5_ref, b5_ref,
                    o_ref):
    for n in range(h_ref.shape[0]):
        h3 = _conv3x3(h_ref[n], w3_ref, b3_ref)
        h4 = _conv3x3(_repad(h3, 384), w4_ref, b4_ref)
        h5 = _conv3x3(_repad(h4, 384), w5_ref, b5_ref).astype(BF16)
        pooled = _maxpool3s2(h5.reshape(13, 16, 256), 6, 6)
        o_ref[n] = pooled.reshape(36, 256).T


def _fc_kernel(h_ref, w_ref, b_ref, o_ref):
    y = lax.dot_general(h_ref[...], w_ref[...], (((1,), (1,)), ((), ())),
                        preferred_element_type=F32)
    o_ref[...] = jnp.maximum(y + b_ref[...], 0.0)


def _band(c):
    i = lax.broadcasted_iota(jnp.int32, (c, c), 0)
    j = lax.broadcasted_iota(jnp.int32, (c, c), 1)
    return (jnp.abs(i - j) <= 2).astype(BF16)


def kernel(x, W1, b1, W2, b2, W3, b3, W4, b4, W5, b5, Wf1, bf1, Wf2, bf2,
           Wf3, bf3):
    B = x.shape[0]
    cp = lambda sem, vmem: pltpu.CompilerParams(
        dimension_semantics=sem, vmem_limit_bytes=vmem)

    GA, GB, GC = 1, 1, 4
    xs = jnp.pad(x, ((0, 0), (0, 0), (2, 2), (2, 30)))
    x6 = xs.reshape(B, 3, 57, 4, 64, 4)
    X2 = x6.transpose(0, 3, 5, 1, 2, 4).reshape(B, 4, 12, 3648)
    X2 = jnp.pad(X2, ((0, 0), (0, 0), (0, 4), (0, 66)))

    W1p = jnp.pad(W1, ((0, 0), (0, 0), (0, 0), (0, 1)))
    W1r = W1p.reshape(96, 3, 11, 3, 4).transpose(2, 3, 4, 1, 0)
    W1k = jnp.pad(W1r.reshape(11, 3, 12, 96),
                  ((0, 0), (0, 0), (0, 4), (0, 32))).reshape(528, 128)
    W1k = W1k.astype(BF16)

    band128 = _band(128)
    h1 = pl.pallas_call(
        _stage_a_kernel,
        grid=(B // GA,),
        in_specs=[
            pl.BlockSpec((GA, 4, 16, 3714), lambda i: (i, 0, 0, 0)),
            pl.BlockSpec((528, 128), lambda i: (0, 0)),
            pl.BlockSpec((1, 128), lambda i: (0, 0)),
            pl.BlockSpec((128, 128), lambda i: (0, 0)),
        ],
        out_specs=pl.BlockSpec((GA, 27, 27, 128), lambda i: (i, 0, 0, 0)),
        out_shape=jax.ShapeDtypeStruct((B, 27, 27, 128), BF16),
        compiler_params=cp(("arbitrary",), 56 * 1024 * 1024),
    )(X2, W1k, jnp.pad(b1, (0, 32)).reshape(1, 128), band128)

    W2k = jnp.pad(W2.transpose(2, 3, 1, 0), ((0, 0), (0, 0), (0, 32), (0, 0)))
    W2k = W2k.reshape(5, 640, 256).astype(BF16)
    band256 = _band(256)
    h2 = pl.pallas_call(
        _stage_b_kernel,
        grid=(B // GB,),
        in_specs=[
            pl.BlockSpec((GB, 27, 27, 128), lambda i: (i, 0, 0, 0)),
            pl.BlockSpec((5, 640, 256), lambda i: (0, 0, 0)),
            pl.BlockSpec((1, 256), lambda i: (0, 0)),
            pl.BlockSpec((256, 256), lambda i: (0, 0)),
        ],
        out_specs=pl.BlockSpec((GB, 16, 16, 256), lambda i: (i, 0, 0, 0)),
        out_shape=jax.ShapeDtypeStruct((B, 16, 16, 256), BF16),
        compiler_params=cp(("arbitrary",), 56 * 1024 * 1024),
    )(h1, W2k, b2.reshape(1, 256), band256)

    W3k = W3.transpose(2, 3, 1, 0).reshape(3, 768, 384).astype(BF16)
    W4k = W4.transpose(2, 3, 1, 0).reshape(3, 1152, 384).astype(BF16)
    W5k = W5.transpose(2, 3, 1, 0).reshape(3, 1152, 256).astype(BF16)
    h5 = pl.pallas_call(
        _stage_c_kernel,
        grid=(B // GC,),
        in_specs=[
            pl.BlockSpec((GC, 16, 16, 256), lambda i: (i, 0, 0, 0)),
            pl.BlockSpec((3, 768, 384), lambda i: (0, 0, 0)),
            pl.BlockSpec((1, 384), lambda i: (0, 0)),
            pl.BlockSpec((3, 1152, 384), lambda i: (0, 0, 0)),
            pl.BlockSpec((1, 384), lambda i: (0, 0)),
            pl.BlockSpec((3, 1152, 256), lambda i: (0, 0, 0)),
            pl.BlockSpec((1, 256), lambda i: (0, 0)),
        ],
        out_specs=pl.BlockSpec((GC, 256, 36), lambda i: (i, 0, 0)),
        out_shape=jax.ShapeDtypeStruct((B, 256, 36), BF16),
        compiler_params=cp(("arbitrary",), 56 * 1024 * 1024),
    )(h2, W3k, b3.reshape(1, 384), W4k, b4.reshape(1, 384), W5k,
      b5.reshape(1, 256))

    hf = h5.reshape(B, 9216).astype(F32)

    def fc(h, W, b, nblk, vmem):
        N, K = W.shape
        return pl.pallas_call(
            _fc_kernel,
            grid=(N // nblk,),
            in_specs=[
                pl.BlockSpec((h.shape[0], K), lambda j: (0, 0)),
                pl.BlockSpec((nblk, K), lambda j: (j, 0)),
                pl.BlockSpec((1, nblk), lambda j: (0, j)),
            ],
            out_specs=pl.BlockSpec((h.shape[0], nblk), lambda j: (0, j)),
            out_shape=jax.ShapeDtypeStruct((h.shape[0], N), F32),
            compiler_params=cp(("arbitrary",), vmem),
        )(h, W, b.reshape(1, N))

    g1 = fc(hf, Wf1, bf1, 512, 56 * 1024 * 1024)
    g2 = fc(g1, Wf2, bf2, 512, 48 * 1024 * 1024)
    out = fc(g2, Wf3, bf3, 1000, 48 * 1024 * 1024)
    return out
